# baseline (speedup 1.0000x reference)
import jax
import jax.numpy as jnp
from jax.experimental import pallas as pl
from jax.experimental.pallas import tpu as pltpu

_TEMP = 0.05
_INV_TEMP = 1.0 / _TEMP
_LOG2E = 1.4426950408889634
_LN2 = 0.6931471805599453
_BLK = 10000


def _wm_loss_kernel(x_ref, t_ref, ft_ref, loss_ref, xn_ref, acc_ref, tv_ref,
                    st_ref):
    i = pl.program_id(0)
    nb = pl.num_programs(0)

    @pl.when(i == 0)
    def _init():
        x = x_ref[...]
        norm = jnp.sqrt(jnp.sum(x * x, axis=1, keepdims=True))
        xn = x * ((_INV_TEMP * _LOG2E) / jnp.maximum(norm, 1e-12))
        xn_ref[...] = xn.astype(jnp.bfloat16)
        acc_ref[...] = jnp.zeros_like(acc_ref)
        tv_ref[...] = jnp.zeros_like(tv_ref)
        st_ref[0, 0] = 0.0

    xn = xn_ref[...]
    ft = ft_ref[0]
    l2 = jax.lax.dot_general(
        xn, ft, (((1,), (0,)), ((), ())),
        preferred_element_type=jnp.float32)
    acc_ref[...] += jnp.sum(jnp.exp2(l2), axis=1, keepdims=True)
    t = t_ref[0]
    tv_ref[...] += jnp.sum(ft.astype(jnp.float32) * t, axis=1,
                           keepdims=True)
    st_ref[0, 0] += jnp.sum(t)

    @pl.when(i == nb - 1)
    def _fin():
        mean_lse = jnp.mean(jnp.log(acc_ref[...]))
        xs = jnp.sum(xn_ref[...].astype(jnp.float32), axis=0,
                     keepdims=True)
        cross = jnp.sum(xs * tv_ref[...].T) * (_LN2 / xn_ref.shape[0])
        loss = st_ref[0, 0] * mean_lse - cross
        loss_ref[...] = jnp.reshape(loss, (1, 1))


def kernel(inputs, targets, features):
    b, d = inputs.shape
    n = features.shape[0]
    nb = n // _BLK
    t3 = targets.reshape(nb, 1, _BLK)
    ft16 = jnp.transpose(features.reshape(nb, _BLK, d),
                         (0, 2, 1)).astype(jnp.bfloat16)
    out = pl.pallas_call(
        _wm_loss_kernel,
        grid=(nb,),
        in_specs=[
            pl.BlockSpec((b, d), lambda i: (0, 0)),
            pl.BlockSpec((1, 1, _BLK), lambda i: (i, 0, 0)),
            pl.BlockSpec((1, d, _BLK), lambda i: (i, 0, 0)),
        ],
        out_specs=pl.BlockSpec((1, 1), lambda i: (0, 0)),
        out_shape=jax.ShapeDtypeStruct((1, 1), jnp.float32),
        scratch_shapes=[
            pltpu.VMEM((b, d), jnp.bfloat16),
            pltpu.VMEM((b, 1), jnp.float32),
            pltpu.VMEM((d, 1), jnp.float32),
            pltpu.SMEM((1, 1), jnp.float32),
        ],
    )(inputs, t3, ft16)
    return out[0, 0]


# speedup vs baseline: 1.2259x; 1.2259x over previous
import jax
import jax.numpy as jnp
from jax.experimental import pallas as pl
from jax.experimental.pallas import tpu as pltpu

_TEMP = 0.05
_INV_TEMP = 1.0 / _TEMP
_LOG2E = 1.4426950408889634
_LN2 = 0.6931471805599453
_BLK = 10000


def _wm_loss_kernel(x_ref, t_ref, f_ref, loss_ref, xn_ref, acc_ref, tv_ref,
                    st_ref):
    i = pl.program_id(0)
    nb = pl.num_programs(0)

    @pl.when(i == 0)
    def _init():
        x = x_ref[...]
        norm = jnp.sqrt(jnp.sum(x * x, axis=1, keepdims=True))
        xn = x * ((_INV_TEMP * _LOG2E) / jnp.maximum(norm, 1e-12))
        xn_ref[...] = xn.astype(jnp.bfloat16)
        acc_ref[...] = jnp.zeros_like(acc_ref)
        tv_ref[...] = jnp.zeros_like(tv_ref)
        st_ref[0, 0] = 0.0

    xn = xn_ref[...]
    f = f_ref[...]
    l2 = jax.lax.dot_general(
        xn, f, (((1,), (1,)), ((), ())),
        preferred_element_type=jnp.float32)
    acc_ref[...] += jnp.sum(jnp.exp2(l2), axis=1, keepdims=True)
    t = t_ref[0]
    tv_ref[...] += jax.lax.dot_general(
        t.astype(jnp.bfloat16), f, (((1,), (0,)), ((), ())),
        preferred_element_type=jnp.float32)
    st_ref[0, 0] += jnp.sum(t)

    @pl.when(i == nb - 1)
    def _fin():
        mean_lse = jnp.mean(jnp.log(acc_ref[...]))
        xs = jnp.sum(xn_ref[...].astype(jnp.float32), axis=0,
                     keepdims=True)
        cross = jnp.sum(xs * tv_ref[...]) * (_LN2 / xn_ref.shape[0])
        loss = st_ref[0, 0] * mean_lse - cross
        loss_ref[...] = jnp.reshape(loss, (1, 1))


def kernel(inputs, targets, features):
    b, d = inputs.shape
    n = features.shape[0]
    nb = n // _BLK
    t3 = targets.reshape(nb, 1, _BLK)
    f16 = features.astype(jnp.bfloat16)
    out = pl.pallas_call(
        _wm_loss_kernel,
        grid=(nb,),
        in_specs=[
            pl.BlockSpec((b, d), lambda i: (0, 0)),
            pl.BlockSpec((1, 1, _BLK), lambda i: (i, 0, 0)),
            pl.BlockSpec((_BLK, d), lambda i: (i, 0)),
        ],
        out_specs=pl.BlockSpec((1, 1), lambda i: (0, 0)),
        out_shape=jax.ShapeDtypeStruct((1, 1), jnp.float32),
        scratch_shapes=[
            pltpu.VMEM((b, d), jnp.bfloat16),
            pltpu.VMEM((b, 1), jnp.float32),
            pltpu.VMEM((1, d), jnp.float32),
            pltpu.SMEM((1, 1), jnp.float32),
        ],
    )(inputs, t3, f16)
    return out[0, 0]
